# baseline (speedup 1.0000x reference)
import functools

import jax
import jax.numpy as jnp
from jax import lax
from jax.experimental import pallas as pl
from jax.experimental.pallas import tpu as pltpu
from jax.experimental.pallas import tpu_sc as plsc

MASK_VALUE = -100000.0
NUM_CORES = 2
NUM_SUBCORES = 16
LANES = 16
D = 128
BUF_ROWS = 64
ROWS_FRAC_CORE0 = 17 / 32


def _make_sc_kernel(B):
    rows_total = B // NUM_SUBCORES
    rows_c0 = int(rows_total * ROWS_FRAC_CORE0) // BUF_ROWS * BUF_ROWS + BUF_ROWS // 2
    rows_by_core = (rows_c0, rows_total - rows_c0)
    rows_max = max(rows_by_core)
    mesh = plsc.VectorSubcoreMesh(
        core_axis_name="c", subcore_axis_name="s", num_cores=NUM_CORES
    )

    @functools.partial(
        pl.kernel,
        out_type=[
            jax.ShapeDtypeStruct((B * D,), jnp.float32),
            jax.ShapeDtypeStruct((B,), jnp.float32),
        ],
        mesh=mesh,
        scratch_types=[
            pltpu.VMEM((D,), jnp.float32),
            pltpu.VMEM((BUF_ROWS * D,), jnp.float32),
            pltpu.VMEM((rows_max,), jnp.float32),
            pltpu.SemaphoreType.DMA,
        ],
    )
    def sc_kernel(mask_hbm, logits_hbm, critic_hbm, mask_v, buf_v, critic_v, sem):
        cid = lax.axis_index("c")
        sid = lax.axis_index("s")

        pltpu.sync_copy(mask_hbm.at[0], mask_v)

        zeros16 = jnp.zeros((LANES,), jnp.float32)
        maskval16 = jnp.full((LANES,), MASK_VALUE, jnp.float32)
        chunks = []
        for c in range(D // LANES):
            m = mask_v[pl.ds(c * LANES, LANES)]
            chunks.append(jnp.where(m == 0.0, zeros16, maskval16))

        def fill_row(r, _):
            for c in range(D // LANES):
                buf_v[pl.ds(r * D + c * LANES, LANES)] = chunks[c]
            return _

        lax.fori_loop(0, BUF_ROWS, fill_row, 0)

        def fill_zero(j, _):
            critic_v[pl.ds(j * LANES, LANES)] = zeros16
            return _

        lax.fori_loop(0, rows_max // LANES, fill_zero, 0)

        def emit_core(core_idx):
            rows_w = rows_by_core[core_idx]
            base = sum(rows_by_core[:core_idx]) * NUM_SUBCORES + sid * rows_w

            def do():
                copies = [
                    pltpu.async_copy(
                        critic_v.at[pl.ds(0, rows_w)],
                        critic_hbm.at[pl.ds(base, rows_w)],
                        sem,
                    )
                ]
                n_full, rem = divmod(rows_w, BUF_ROWS)
                for d_i in range(n_full):
                    dst = logits_hbm.at[
                        pl.ds((base + d_i * BUF_ROWS) * D, BUF_ROWS * D)
                    ]
                    copies.append(pltpu.async_copy(buf_v, dst, sem))
                if rem:
                    dst = logits_hbm.at[
                        pl.ds((base + n_full * BUF_ROWS) * D, rem * D)
                    ]
                    copies.append(
                        pltpu.async_copy(buf_v.at[pl.ds(0, rem * D)], dst, sem)
                    )
                for cp in copies:
                    cp.wait()

            return do

        pl.when(cid == 0)(emit_core(0))
        pl.when(cid == 1)(emit_core(1))

    return sc_kernel


def kernel(inputs, all_action_masks):
    B = inputs.shape[0]
    logits_flat, critic_flat = _make_sc_kernel(B)(all_action_masks)
    return logits_flat.reshape(B, D), critic_flat.reshape(B, 1)


# speedup vs baseline: 1.0270x; 1.0270x over previous
import functools

import jax
import jax.numpy as jnp
from jax import lax
from jax.experimental import pallas as pl
from jax.experimental.pallas import tpu as pltpu
from jax.experimental.pallas import tpu_sc as plsc

MASK_VALUE = -100000.0
NUM_CORES = 2
NUM_SUBCORES = 16
NUM_WORKERS = NUM_CORES * NUM_SUBCORES
LANES = 16
D = 128
BUF_ROWS = 64


def _make_sc_kernel(B):
    rows_per_w = B // NUM_WORKERS
    n_dma = rows_per_w // BUF_ROWS
    mesh = plsc.VectorSubcoreMesh(
        core_axis_name="c", subcore_axis_name="s", num_cores=NUM_CORES
    )

    @functools.partial(
        pl.kernel,
        out_type=[
            jax.ShapeDtypeStruct((B * D,), jnp.float32),
            jax.ShapeDtypeStruct((B,), jnp.float32),
        ],
        mesh=mesh,
        scratch_types=[
            pltpu.VMEM((D,), jnp.float32),
            pltpu.VMEM((BUF_ROWS * D,), jnp.float32),
            pltpu.VMEM((rows_per_w,), jnp.float32),
            pltpu.SemaphoreType.DMA,
        ],
    )
    def sc_kernel(mask_hbm, logits_hbm, critic_hbm, mask_v, buf_v, critic_v, sem):
        cid = lax.axis_index("c")
        sid = lax.axis_index("s")
        wid = sid * NUM_CORES + cid
        base = wid * rows_per_w

        pltpu.sync_copy(mask_hbm.at[0], mask_v)

        zeros16 = jnp.zeros((LANES,), jnp.float32)
        maskval16 = jnp.full((LANES,), MASK_VALUE, jnp.float32)
        chunks = []
        for c in range(D // LANES):
            m = mask_v[pl.ds(c * LANES, LANES)]
            chunks.append(jnp.where(m == 0.0, zeros16, maskval16))

        def fill_row(r, _):
            for c in range(D // LANES):
                buf_v[pl.ds(r * D + c * LANES, LANES)] = chunks[c]
            return _

        lax.fori_loop(0, BUF_ROWS, fill_row, 0)

        def fill_zero(j, _):
            critic_v[pl.ds(j * LANES, LANES)] = zeros16
            return _

        lax.fori_loop(0, rows_per_w // LANES, fill_zero, 0)

        copies = []
        for d_i in range(n_dma):
            dst = logits_hbm.at[pl.ds((base + d_i * BUF_ROWS) * D, BUF_ROWS * D)]
            copies.append(pltpu.async_copy(buf_v, dst, sem))
        copies.append(
            pltpu.async_copy(critic_v, critic_hbm.at[pl.ds(base, rows_per_w)], sem)
        )
        for cp in copies:
            cp.wait()

    return sc_kernel


def kernel(inputs, all_action_masks):
    B = inputs.shape[0]
    logits_flat, critic_flat = _make_sc_kernel(B)(all_action_masks)
    return logits_flat.reshape(B, D), critic_flat.reshape(B, 1)
